# baseline (speedup 1.0000x reference)
import functools
import math

import jax
import jax.numpy as jnp
from jax import lax
from jax.experimental import pallas as pl
from jax.experimental.pallas import tpu as pltpu
from jax.experimental.pallas import tpu_sc as plsc

SMOOTH = 0.1
CONF = 1.0 - SMOOTH
NC, NS, L = 2, 16, 16
NW = NC * NS
R = 40
TCROWS = 600
RCH = 200


def _make_tc_part(batch, v, toks):
    nr = TCROWS // RCH

    def tc_kernel(q_ref, t_ref, m_ref, a_ref, g_ref):
        r = pl.program_id(1)
        x = q_ref[0]
        tb = t_ref[0, 0]
        mb = m_ref[0, 0]
        s = jnp.sum(x, axis=0)
        rid = lax.broadcasted_iota(jnp.int32, (RCH, toks), 0) + r * RCH
        gmask = rid == tb[None, :]
        g = jnp.sum(jnp.where(gmask, x, 0.0), axis=0)
        wf = jnp.where(mb > 0, 1.0, 0.0)

        @pl.when(r == 0)
        def _():
            a_ref[0, 0] = wf * s
            g_ref[0, 0] = wf * g

        @pl.when(r > 0)
        def _():
            a_ref[0, 0] = a_ref[0, 0] + wf * s
            g_ref[0, 0] = g_ref[0, 0] + wf * g

    return pl.pallas_call(
        tc_kernel,
        grid=(batch, nr),
        in_specs=[
            pl.BlockSpec((1, RCH, toks), lambda b, r: (b, r, 0)),
            pl.BlockSpec((1, 1, toks), lambda b, r: (b, 0, 0)),
            pl.BlockSpec((1, 1, toks), lambda b, r: (b, 0, 0)),
        ],
        out_specs=[
            pl.BlockSpec((1, 1, toks), lambda b, r: (b, 0, 0)),
            pl.BlockSpec((1, 1, toks), lambda b, r: (b, 0, 0)),
        ],
        out_shape=[
            jax.ShapeDtypeStruct((batch, 1, toks), jnp.float32),
            jax.ShapeDtypeStruct((batch, 1, toks), jnp.float32),
        ],
    )


def _make_sc_part(batch, v, toks):
    stripe = toks // NW
    kv = stripe // L
    scrows = v - TCROWS
    cpb = scrows // R
    nch = batch * cpb
    assert nch % 2 == 0 and scrows % R == 0 and R % 8 == 0
    eps = SMOOTH / (v - 1)
    c_const = CONF * math.log(CONF) + (v - 1) * eps * math.log(eps)

    mesh = plsc.VectorSubcoreMesh(
        core_axis_name="c", subcore_axis_name="s",
        num_cores=NC, num_subcores=NS)

    @functools.partial(
        pl.kernel,
        out_type=(
            jax.ShapeDtypeStruct((NW * L,), jnp.float32),
            jax.ShapeDtypeStruct((NW * L,), jnp.float32),
        ),
        mesh=mesh,
        compiler_params=pltpu.CompilerParams(needs_layout_passes=False),
        scratch_types=[
            pltpu.VMEM((batch * stripe,), jnp.int32),
            pltpu.VMEM((batch * stripe,), jnp.int32),
            pltpu.VMEM((R, stripe), jnp.float32),
            pltpu.VMEM((R, stripe), jnp.float32),
            pltpu.VMEM((L,), jnp.float32),
            pltpu.VMEM((L,), jnp.float32),
            pltpu.SemaphoreType.DMA,
            pltpu.SemaphoreType.DMA,
        ],
    )
    def k(q_hbm, tgt_hbm, msk_hbm, out_hbm, out2_hbm,
          mvec, tvec, buf0, buf1, stage, stage2, sem0, sem1):
        wid = lax.axis_index("s") * NC + lax.axis_index("c")
        col0 = wid * stripe
        iota = lax.iota(jnp.int32, L)

        def bc(x, dtype):
            return lax.broadcast(jnp.asarray(x, dtype), (L,))

        for b in range(batch):
            pltpu.sync_copy(msk_hbm.at[pl.ds(b * toks + col0, stripe)],
                            mvec.at[pl.ds(b * stripe, stripe)])
            pltpu.sync_copy(tgt_hbm.at[pl.ds(b * toks + col0, stripe)],
                            tvec.at[pl.ds(b * stripe, stripe)])

        zv = jnp.zeros((L,), jnp.float32)

        nacc = zv
        for kk in range(batch * kv):
            nacc = nacc + jnp.where(mvec[pl.ds(kk * L, L)] > 0, 1.0, 0.0)

        def start(ci, buf, sem):
            b = ci // cpb
            c = ci - b * cpb
            src = q_hbm.at[pl.ds(b * v + TCROWS + c * R, R),
                           pl.ds(col0, stripe)]
            return pltpu.async_copy(src, buf, sem)

        def process(ci, buf, sem, carry):
            pltpu.make_async_copy(
                q_hbm.at[pl.ds(0, R), pl.ds(col0, stripe)], buf, sem).wait()
            b = ci // cpb
            c = ci - b * cpb
            accs = carry

            def row_body(r2, cc):
                cc = list(cc)
                for rr in range(4):
                    for kk in range(kv):
                        cc[kk] = cc[kk] + buf[r2 * 4 + rr, pl.ds(kk * L, L)]
                return tuple(cc)

            local = lax.fori_loop(0, R // 4, row_body, tuple([zv] * kv))

            out = []
            for kk in range(kv):
                mk = mvec[pl.ds(b * stripe + kk * L, L)]
                wf = jnp.where(mk > 0, 1.0, 0.0)
                tk = tvec[pl.ds(b * stripe + kk * L, L)]
                rowidx = tk - bc(TCROWS + c * R, jnp.int32)
                inb = (rowidx >= 0) & (rowidx < R)
                srow = jnp.where(inb, rowidx, 0)
                val = plsc.load_gather(buf, [srow, kk * L + iota])
                g_add = jnp.where(inb, wf * val, 0.0)
                s_k = accs[kk] + wf * local[kk]
                g_k = accs[kv + kk] + g_add
                out.append((s_k, g_k))
            return tuple(x[0] for x in out) + tuple(x[1] for x in out)

        carry = tuple([zv] * (2 * kv))
        start(0, buf0, sem0)
        start(1, buf1, sem1)

        def pair_body(u, carry):
            ci0 = u * 2
            carry = process(ci0, buf0, sem0, carry)
            start(ci0 + 2, buf0, sem0)
            carry = process(ci0 + 1, buf1, sem1, carry)
            start(ci0 + 3, buf1, sem1)
            return carry

        carry = lax.fori_loop(0, nch // 2 - 1, pair_body, carry)
        carry = process(nch - 2, buf0, sem0, carry)
        carry = process(nch - 1, buf1, sem1, carry)

        stot = carry[0]
        for kk in range(1, kv):
            stot = stot + carry[kk]
        gtot = carry[kv]
        for kk in range(1, kv):
            gtot = gtot + carry[kv + kk]

        numer = -eps * stot - (CONF - eps) * gtot + c_const * nacc
        stage[...] = numer
        pltpu.sync_copy(stage, out_hbm.at[pl.ds(wid * L, L)])
        stage2[...] = nacc
        pltpu.sync_copy(stage2, out2_hbm.at[pl.ds(wid * L, L)])

    return k


def kernel(prediction, target, mask):
    batch, toks, v = prediction.shape
    q3 = prediction.transpose(0, 2, 1)
    q2 = q3.reshape(batch * v, toks)
    t2 = target.astype(jnp.int32)
    m2 = mask.astype(jnp.int32)
    t1 = t2.reshape(-1)
    m1 = m2.reshape(-1)
    eps = SMOOTH / (v - 1)
    numer_sc, cnt = _make_sc_part(batch, v, toks)(q2, t1, m1)
    a_tc, g_tc = _make_tc_part(batch, v, toks)(
        q3, t2.reshape(batch, 1, toks), m2.reshape(batch, 1, toks))
    numer = (jnp.sum(numer_sc)
             - eps * jnp.sum(a_tc)
             - (CONF - eps) * jnp.sum(g_tc))
    return numer / jnp.sum(cnt)


# speedup vs baseline: 1.0793x; 1.0793x over previous
import functools
import math

import jax
import jax.numpy as jnp
from jax import lax
from jax.experimental import pallas as pl
from jax.experimental.pallas import tpu as pltpu
from jax.experimental.pallas import tpu_sc as plsc

SMOOTH = 0.1
CONF = 1.0 - SMOOTH
NC, NS, L = 2, 16, 16
NW = NC * NS
R = 40
TCROWS = 840
RCH = 280


def _make_tc_part(batch, v, toks):
    nr = TCROWS // RCH

    def tc_kernel(q_ref, t_ref, m_ref, a_ref, g_ref):
        r = pl.program_id(1)
        x = q_ref[0]
        tb = t_ref[0, 0]
        mb = m_ref[0, 0]
        s = jnp.sum(x, axis=0)
        rid = lax.broadcasted_iota(jnp.int32, (RCH, toks), 0) + r * RCH
        gmask = rid == tb[None, :]
        g = jnp.sum(jnp.where(gmask, x, 0.0), axis=0)
        wf = jnp.where(mb > 0, 1.0, 0.0)

        @pl.when(r == 0)
        def _():
            a_ref[0, 0] = wf * s
            g_ref[0, 0] = wf * g

        @pl.when(r > 0)
        def _():
            a_ref[0, 0] = a_ref[0, 0] + wf * s
            g_ref[0, 0] = g_ref[0, 0] + wf * g

    return pl.pallas_call(
        tc_kernel,
        grid=(batch, nr),
        in_specs=[
            pl.BlockSpec((1, RCH, toks), lambda b, r: (b, r, 0)),
            pl.BlockSpec((1, 1, toks), lambda b, r: (b, 0, 0)),
            pl.BlockSpec((1, 1, toks), lambda b, r: (b, 0, 0)),
        ],
        out_specs=[
            pl.BlockSpec((1, 1, toks), lambda b, r: (b, 0, 0)),
            pl.BlockSpec((1, 1, toks), lambda b, r: (b, 0, 0)),
        ],
        out_shape=[
            jax.ShapeDtypeStruct((batch, 1, toks), jnp.float32),
            jax.ShapeDtypeStruct((batch, 1, toks), jnp.float32),
        ],
    )


def _make_sc_part(batch, v, toks):
    stripe = toks // NW
    kv = stripe // L
    scrows = v - TCROWS
    cpb = scrows // R
    nch = batch * cpb
    assert nch % 2 == 0 and scrows % R == 0 and R % 8 == 0
    eps = SMOOTH / (v - 1)
    c_const = CONF * math.log(CONF) + (v - 1) * eps * math.log(eps)

    mesh = plsc.VectorSubcoreMesh(
        core_axis_name="c", subcore_axis_name="s",
        num_cores=NC, num_subcores=NS)

    @functools.partial(
        pl.kernel,
        out_type=(
            jax.ShapeDtypeStruct((NW * L,), jnp.float32),
            jax.ShapeDtypeStruct((NW * L,), jnp.float32),
        ),
        mesh=mesh,
        compiler_params=pltpu.CompilerParams(needs_layout_passes=False),
        scratch_types=[
            pltpu.VMEM((batch * stripe,), jnp.int32),
            pltpu.VMEM((batch * stripe,), jnp.int32),
            pltpu.VMEM((R, stripe), jnp.float32),
            pltpu.VMEM((R, stripe), jnp.float32),
            pltpu.VMEM((L,), jnp.float32),
            pltpu.VMEM((L,), jnp.float32),
            pltpu.SemaphoreType.DMA,
            pltpu.SemaphoreType.DMA,
        ],
    )
    def k(q_hbm, tgt_hbm, msk_hbm, out_hbm, out2_hbm,
          mvec, tvec, buf0, buf1, stage, stage2, sem0, sem1):
        wid = lax.axis_index("s") * NC + lax.axis_index("c")
        col0 = wid * stripe
        iota = lax.iota(jnp.int32, L)

        def bc(x, dtype):
            return lax.broadcast(jnp.asarray(x, dtype), (L,))

        for b in range(batch):
            pltpu.sync_copy(msk_hbm.at[pl.ds(b * toks + col0, stripe)],
                            mvec.at[pl.ds(b * stripe, stripe)])
            pltpu.sync_copy(tgt_hbm.at[pl.ds(b * toks + col0, stripe)],
                            tvec.at[pl.ds(b * stripe, stripe)])

        zv = jnp.zeros((L,), jnp.float32)

        nacc = zv
        for kk in range(batch * kv):
            nacc = nacc + jnp.where(mvec[pl.ds(kk * L, L)] > 0, 1.0, 0.0)

        def start(ci, buf, sem):
            b = ci // cpb
            c = ci - b * cpb
            src = q_hbm.at[pl.ds(b * v + TCROWS + c * R, R),
                           pl.ds(col0, stripe)]
            return pltpu.async_copy(src, buf, sem)

        def process(ci, buf, sem, carry):
            pltpu.make_async_copy(
                q_hbm.at[pl.ds(0, R), pl.ds(col0, stripe)], buf, sem).wait()
            b = ci // cpb
            c = ci - b * cpb
            accs = carry

            def row_body(r2, cc):
                cc = list(cc)
                for rr in range(4):
                    for kk in range(kv):
                        cc[kk] = cc[kk] + buf[r2 * 4 + rr, pl.ds(kk * L, L)]
                return tuple(cc)

            local = lax.fori_loop(0, R // 4, row_body, tuple([zv] * kv))

            out = []
            for kk in range(kv):
                mk = mvec[pl.ds(b * stripe + kk * L, L)]
                wf = jnp.where(mk > 0, 1.0, 0.0)
                tk = tvec[pl.ds(b * stripe + kk * L, L)]
                rowidx = tk - bc(TCROWS + c * R, jnp.int32)
                inb = (rowidx >= 0) & (rowidx < R)
                srow = jnp.where(inb, rowidx, 0)
                val = plsc.load_gather(buf, [srow, kk * L + iota])
                g_add = jnp.where(inb, wf * val, 0.0)
                s_k = accs[kk] + wf * local[kk]
                g_k = accs[kv + kk] + g_add
                out.append((s_k, g_k))
            return tuple(x[0] for x in out) + tuple(x[1] for x in out)

        carry = tuple([zv] * (2 * kv))
        start(0, buf0, sem0)
        start(1, buf1, sem1)

        def pair_body(u, carry):
            ci0 = u * 2
            carry = process(ci0, buf0, sem0, carry)
            start(ci0 + 2, buf0, sem0)
            carry = process(ci0 + 1, buf1, sem1, carry)
            start(ci0 + 3, buf1, sem1)
            return carry

        carry = lax.fori_loop(0, nch // 2 - 1, pair_body, carry)
        carry = process(nch - 2, buf0, sem0, carry)
        carry = process(nch - 1, buf1, sem1, carry)

        stot = carry[0]
        for kk in range(1, kv):
            stot = stot + carry[kk]
        gtot = carry[kv]
        for kk in range(1, kv):
            gtot = gtot + carry[kv + kk]

        numer = -eps * stot - (CONF - eps) * gtot + c_const * nacc
        stage[...] = numer
        pltpu.sync_copy(stage, out_hbm.at[pl.ds(wid * L, L)])
        stage2[...] = nacc
        pltpu.sync_copy(stage2, out2_hbm.at[pl.ds(wid * L, L)])

    return k


def kernel(prediction, target, mask):
    batch, toks, v = prediction.shape
    q3 = prediction.transpose(0, 2, 1)
    q2 = q3.reshape(batch * v, toks)
    t2 = target.astype(jnp.int32)
    m2 = mask.astype(jnp.int32)
    t1 = t2.reshape(-1)
    m1 = m2.reshape(-1)
    eps = SMOOTH / (v - 1)
    numer_sc, cnt = _make_sc_part(batch, v, toks)(q2, t1, m1)
    a_tc, g_tc = _make_tc_part(batch, v, toks)(
        q3, t2.reshape(batch, 1, toks), m2.reshape(batch, 1, toks))
    numer = (jnp.sum(numer_sc)
             - eps * jnp.sum(a_tc)
             - (CONF - eps) * jnp.sum(g_tc))
    return numer / jnp.sum(cnt)
